# baseline (speedup 1.0000x reference)
import functools

import jax
import jax.numpy as jnp
from jax import lax
from jax.experimental import pallas as pl
from jax.experimental.pallas import tpu as pltpu
from jax.experimental.pallas import tpu_sc as plsc

N = 10000
E = 320000
D = 128
DE = 16
H = 128

NC = 2
NS = 16
NW = NC * NS
EPT = E // NW
C = 80
NCHUNK = EPT // C
RPT = N // NS
LANES = 16
VPR = H // LANES

_mesh = plsc.VectorSubcoreMesh(core_axis_name="c", subcore_axis_name="s")

f32 = jnp.float32
i32 = jnp.int32
bf16 = jnp.bfloat16


NBUF = 5
E1 = 192000
E2 = E - E1
CB = 80


def _make_gather_combine(ne):
  ept = ne // NW
  nchunk = ept // CB
  assert nchunk % NBUF == 0

  @functools.partial(
      pl.kernel,
      out_type=jax.ShapeDtypeStruct((ne, H), f32),
      mesh=_mesh,
      compiler_params=pltpu.CompilerParams(use_tc_tiling_on_sc=True),
      scratch_types=(
          [pltpu.VMEM((ept,), i32), pltpu.VMEM((ept,), i32)]
          + [pltpu.VMEM((CB, H), f32)] * NBUF
          + [pltpu.SemaphoreType.DMA] * (3 * NBUF)
      ),
  )
  def gather_combine(src_hbm, dst_hbm, xs_hbm, xdn_hbm, g_hbm,
                     idx_s, idx_d, *rest):
    bufs = rest[:NBUF]
    gsem = rest[NBUF:2 * NBUF]
    asem = rest[2 * NBUF:3 * NBUF]
    ssem = rest[3 * NBUF:]
    cid = lax.axis_index("c")
    sid = lax.axis_index("s")
    wid = sid * NC + cid
    base = wid * ept

    pltpu.sync_copy(src_hbm.at[pl.ds(base, ept)], idx_s)
    pltpu.sync_copy(dst_hbm.at[pl.ds(base, ept)], idx_d)

    def g_issue(c, s):
      pltpu.async_copy(xs_hbm.at[idx_s.at[pl.ds(c * CB, CB)]], bufs[s],
                       gsem[s])

    def g_wait(s):
      pltpu.make_async_copy(xs_hbm.at[pl.ds(0, CB)], bufs[s], gsem[s]).wait()

    def a_issue(c, s):
      pltpu.async_copy(xdn_hbm.at[idx_d.at[pl.ds(c * CB, CB)]], bufs[s],
                       asem[s], add=True)

    def a_wait(s):
      pltpu.make_async_copy(xdn_hbm.at[pl.ds(0, CB)], bufs[s], asem[s]).wait()

    def s_issue(c, s):
      pltpu.async_copy(bufs[s], g_hbm.at[pl.ds(base + c * CB, CB)], ssem[s])

    def s_wait(s):
      pltpu.make_async_copy(bufs[s], g_hbm.at[pl.ds(base, CB)], ssem[s]).wait()

    for c in range(NBUF - 1):
      g_issue(c, c)
    for c in range(2):
      g_wait(c)
      a_issue(c, c)

    def outer(jj, carry):
      for b in range(NBUF):
        j = jj * NBUF + b

        @pl.when(j + 4 < nchunk)
        def _():
          s2 = (b + 4) % NBUF

          @pl.when(j >= 1)
          def _():
            s_wait(s2)

          g_issue(j + 4, s2)

        @pl.when(j + 2 < nchunk)
        def _():
          s1 = (b + 2) % NBUF
          g_wait(s1)
          a_issue(j + 2, s1)

        a_wait(b)
        s_issue(j, b)
      return carry

    lax.fori_loop(0, nchunk // NBUF, outer, 0)
    for s in range(NBUF):
      s_wait(s)

  return gather_combine


_sc_gather_combine_1 = _make_gather_combine(E1)
_sc_gather_combine_2 = _make_gather_combine(E2)


CS = LANES
NCHS = EPT // CS
NZ = N // CS


@functools.partial(
    pl.kernel,
    out_type=(jax.ShapeDtypeStruct((N, H), f32),
              jax.ShapeDtypeStruct((N, H), f32)),
    mesh=_mesh,
    compiler_params=pltpu.CompilerParams(use_tc_tiling_on_sc=True),
    scratch_types=(
        [pltpu.VMEM((EPT,), i32), pltpu.VMEM((EPT,), i32),
         pltpu.VMEM((EPT,), f32), pltpu.VMEM_SHARED((N, H), f32)]
        + [pltpu.VMEM((CS, H), f32)] * NBUF
        + [pltpu.SemaphoreType.DMA] * (2 * NBUF)
    ),
)
def _sc_scatter(src_hbm, dst_hbm, alpha_hbm, x_hbm, p0_hbm, p1_hbm,
                idx_s, idx_d, aall, acc, *rest):
  rows = rest[:NBUF]
  gsem = rest[NBUF:2 * NBUF]
  csem = rest[2 * NBUF:]
  cid = lax.axis_index("c")
  sid = lax.axis_index("s")
  wid = sid * NC + cid
  base = wid * EPT

  pltpu.sync_copy(src_hbm.at[pl.ds(base, EPT)], idx_s)
  pltpu.sync_copy(dst_hbm.at[pl.ds(base, EPT)], idx_d)
  pltpu.sync_copy(alpha_hbm.at[pl.ds(base, EPT)], aall)

  def zrow(k, c2):
    for v in range(VPR):
      rows[0][k, pl.ds(v * LANES, LANES)] = jnp.zeros((LANES,), f32)
    return c2

  lax.fori_loop(0, CS, zrow, 0)

  def zchunk(ci, c2):
    @pl.when(ci % NS == sid)
    def _():
      pltpu.sync_copy(rows[0], acc.at[pl.ds(ci * CS, CS)])
    return c2

  lax.fori_loop(0, NZ, zchunk, 0)
  plsc.subcore_barrier()

  def g_issue(c, s):
    iv = idx_d[pl.ds(c * CS, CS)]
    pltpu.async_copy(x_hbm.at[iv], rows[s], gsem[s])

  def g_wait(s):
    pltpu.make_async_copy(x_hbm.at[pl.ds(0, CS)], rows[s], gsem[s]).wait()

  def c_issue(c, s):
    iv = idx_s[pl.ds(c * CS, CS)]
    pltpu.async_copy(rows[s], acc.at[iv], csem[s], add=True)

  def c_wait(s):
    pltpu.make_async_copy(rows[s], acc.at[pl.ds(0, CS)], csem[s]).wait()

  for c in range(NBUF - 1):
    g_issue(c, c)

  def outer(jj, carry):
    for b in range(NBUF):
      j = jj * NBUF + b

      @pl.when(j + 4 < NCHS)
      def _():
        s2 = (b + 4) % NBUF

        @pl.when(j >= 1)
        def _():
          c_wait(s2)

        g_issue(j + 4, s2)

      g_wait(b)
      nav = -aall[pl.ds(j * CS, CS)]
      for t in range(CS):
        a = nav[t]
        for v in range(VPR):
          sl = pl.ds(v * LANES, LANES)
          rows[b][t, sl] = rows[b][t, sl] * a
      c_issue(j, b)
    return carry

  lax.fori_loop(0, NCHS // NBUF, outer, 0)
  for s in range(NBUF):
    c_wait(s)
  plsc.subcore_barrier()

  def drain(ci, c2):
    @pl.when(ci % NS == sid)
    def _():
      sl = pl.ds(ci * CS, CS)

      @pl.when(cid == 0)
      def _():
        pltpu.sync_copy(acc.at[sl], p0_hbm.at[sl])

      @pl.when(cid == 1)
      def _():
        pltpu.sync_copy(acc.at[sl], p1_hbm.at[sl])
    return c2

  lax.fori_loop(0, NZ, drain, 0)


RB = 2000
NRB = N // RB
EB = 8000
NEB = E // EB
SRows = 2500


def _p1_body(x_ref, w1t_ref, w2t_ref, xs_ref, xdn_ref):
  xb = x_ref[...]
  xs_ref[...] = jnp.dot(xb, w1t_ref[...], preferred_element_type=f32)
  xdn_ref[...] = jnp.dot(xb, w2t_ref[...], preferred_element_type=f32)


_p1 = pl.pallas_call(
    _p1_body,
    grid=(NRB,),
    in_specs=[
        pl.BlockSpec((RB, D), lambda i: (i, 0)),
        pl.BlockSpec((D, H), lambda i: (0, 0)),
        pl.BlockSpec((D, H), lambda i: (0, 0)),
    ],
    out_specs=[
        pl.BlockSpec((RB, H), lambda i: (i, 0)),
        pl.BlockSpec((RB, H), lambda i: (i, 0)),
    ],
    out_shape=[
        jax.ShapeDtypeStruct((N, H), f32),
        jax.ShapeDtypeStruct((N, H), f32),
    ],
)


def _p2_body(g_ref, ea_ref, wht_ref, bias_ref, we_ref, s_ref):
  z = (g_ref[...]
       + jnp.dot(ea_ref[...], wht_ref[...], preferred_element_type=f32)
       + bias_ref[...])
  z = jnp.maximum(z, 0.0)
  s = jnp.sum(z * we_ref[...], axis=1)
  s_ref[...] = s.reshape(1, 1, EB)


def _make_p2(ne):
  neb = ne // EB
  return pl.pallas_call(
      _p2_body,
      grid=(neb,),
      in_specs=[
          pl.BlockSpec((EB, H), lambda i: (i, 0)),
          pl.BlockSpec((EB, DE), lambda i: (i, 0)),
          pl.BlockSpec((DE, H), lambda i: (0, 0)),
          pl.BlockSpec((1, H), lambda i: (0, 0)),
          pl.BlockSpec((1, H), lambda i: (0, 0)),
      ],
      out_specs=pl.BlockSpec((1, 1, EB), lambda i: (i, 0, 0)),
      out_shape=jax.ShapeDtypeStruct((neb, 1, EB), f32),
  )


_p2_1 = _make_p2(E1)
_p2_2 = _make_p2(E2)
SR1 = E1 // 128
SR2 = E2 // 128


def _p3_body(s1_ref, s2_ref, a_ref):
  s1 = s1_ref[...]
  s2 = s2_ref[...]
  m = jnp.maximum(jnp.max(s1), jnp.max(s2))
  e1 = jnp.exp(s1 - m)
  e2 = jnp.exp(s2 - m)
  z = jnp.sum(e1) + jnp.sum(e2)
  a_ref[pl.ds(0, SR1), :] = e1 / z
  a_ref[pl.ds(SR1, SR2), :] = e2 / z


_p3 = pl.pallas_call(
    _p3_body,
    out_shape=jax.ShapeDtypeStruct((SRows, 128), f32),
)


def _p4_body(x_ref, p0_ref, p1_ref, ftt_ref, ftb_ref, o_ref):
  o1 = x_ref[...] + p0_ref[...] + p1_ref[...]
  o_ref[...] = (o1 + jnp.dot(o1, ftt_ref[...], preferred_element_type=f32)
                + ftb_ref[...])


_p4 = pl.pallas_call(
    _p4_body,
    grid=(NRB,),
    in_specs=[
        pl.BlockSpec((RB, D), lambda i: (i, 0)),
        pl.BlockSpec((RB, D), lambda i: (i, 0)),
        pl.BlockSpec((RB, D), lambda i: (i, 0)),
        pl.BlockSpec((D, D), lambda i: (0, 0)),
        pl.BlockSpec((1, D), lambda i: (0, 0)),
    ],
    out_specs=pl.BlockSpec((RB, D), lambda i: (i, 0)),
    out_shape=jax.ShapeDtypeStruct((N, D), f32),
)


def kernel(x, edge_index, edge_attr, Wh_w, Wh_b, Wn_w, Wn_b, w_e, ft_w, ft_b):
  src = edge_index[0]
  dst = edge_index[1]
  w1t = Wn_w[:, :D].T
  w2t = Wn_w[:, D:].T
  wht = Wh_w.T
  bias = (Wh_b + Wn_b).reshape(1, H)
  ftt = ft_w.T
  ftb = ft_b.reshape(1, D)

  xs, xdn = _p1(x, w1t, w2t)
  g1 = _sc_gather_combine_1(src[:E1], dst[:E1], xs, xdn)
  g2 = _sc_gather_combine_2(src[E1:], dst[E1:], xs, xdn)
  s1 = _p2_1(g1, edge_attr[:E1], wht, bias, w_e.reshape(1, H))
  s2 = _p2_2(g2, edge_attr[E1:], wht, bias, w_e.reshape(1, H))
  alpha2 = _p3(s1.reshape(SR1, 128), s2.reshape(SR2, 128))
  alpha = alpha2.reshape(E)
  part0, part1 = _sc_scatter(src, dst, alpha, x)
  out = _p4(x, part0, part1, ftt, ftb)
  return out, alpha


# speedup vs baseline: 1.0839x; 1.0839x over previous
import functools

import jax
import jax.numpy as jnp
from jax import lax
from jax.experimental import pallas as pl
from jax.experimental.pallas import tpu as pltpu
from jax.experimental.pallas import tpu_sc as plsc

N = 10000
E = 320000
D = 128
DE = 16
H = 128

NC = 2
NS = 16
NW = NC * NS
EPT = E // NW
C = 80
NCHUNK = EPT // C
RPT = N // NS
LANES = 16
VPR = H // LANES

_mesh = plsc.VectorSubcoreMesh(core_axis_name="c", subcore_axis_name="s")

f32 = jnp.float32
i32 = jnp.int32
bf16 = jnp.bfloat16


NBUF = 5
assert NCHUNK % NBUF == 0


@functools.partial(
    pl.kernel,
    out_type=jax.ShapeDtypeStruct((E, H), f32),
    mesh=_mesh,
    scratch_types=(
        [pltpu.VMEM((EPT,), i32), pltpu.VMEM((EPT,), i32)]
        + [pltpu.VMEM((C, H), f32)] * NBUF
        + [pltpu.SemaphoreType.DMA] * (3 * NBUF)
    ),
)
def _sc_gather_combine(src_hbm, dst_hbm, xs_hbm, xdn_hbm, g_hbm,
                       idx_s, idx_d, *rest):
  bufs = rest[:NBUF]
  gsem = rest[NBUF:2 * NBUF]
  asem = rest[2 * NBUF:3 * NBUF]
  ssem = rest[3 * NBUF:]
  cid = lax.axis_index("c")
  sid = lax.axis_index("s")
  wid = sid * NC + cid
  base = wid * EPT

  pltpu.sync_copy(src_hbm.at[pl.ds(base, EPT)], idx_s)
  pltpu.sync_copy(dst_hbm.at[pl.ds(base, EPT)], idx_d)

  def g_issue(c, s):
    pltpu.async_copy(xs_hbm.at[idx_s.at[pl.ds(c * C, C)]], bufs[s], gsem[s])

  def g_wait(s):
    pltpu.make_async_copy(xs_hbm.at[pl.ds(0, C)], bufs[s], gsem[s]).wait()

  def a_issue(c, s):
    pltpu.async_copy(xdn_hbm.at[idx_d.at[pl.ds(c * C, C)]], bufs[s], asem[s],
                     add=True)

  def a_wait(s):
    pltpu.make_async_copy(xdn_hbm.at[pl.ds(0, C)], bufs[s], asem[s]).wait()

  def s_issue(c, s):
    pltpu.async_copy(bufs[s], g_hbm.at[pl.ds(base + c * C, C)], ssem[s])

  def s_wait(s):
    pltpu.make_async_copy(bufs[s], g_hbm.at[pl.ds(base, C)], ssem[s]).wait()

  for c in range(NBUF - 1):
    g_issue(c, c)
  for c in range(2):
    g_wait(c)
    a_issue(c, c)

  def outer(jj, carry):
    for b in range(NBUF):
      j = jj * NBUF + b

      @pl.when(j + 4 < NCHUNK)
      def _():
        s2 = (b + 4) % NBUF

        @pl.when(j >= 1)
        def _():
          s_wait(s2)

        g_issue(j + 4, s2)

      @pl.when(j + 2 < NCHUNK)
      def _():
        s1 = (b + 2) % NBUF
        g_wait(s1)
        a_issue(j + 2, s1)

      a_wait(b)
      s_issue(j, b)
    return carry

  lax.fori_loop(0, NCHUNK // NBUF, outer, 0)
  for s in range(NBUF):
    s_wait(s)


CS = 40
NCHS = EPT // CS
NZ = N // CS


@functools.partial(
    pl.kernel,
    out_type=(jax.ShapeDtypeStruct((N, H), f32),
              jax.ShapeDtypeStruct((N, H), f32)),
    mesh=_mesh,
    scratch_types=(
        [pltpu.VMEM((EPT,), i32), pltpu.VMEM_SHARED((N, H), f32)]
        + [pltpu.VMEM((CS, H), f32)] * NBUF
        + [pltpu.VMEM((48,), f32)] * NBUF
        + [pltpu.VMEM((CS,), i32)] * NBUF
        + [pltpu.SemaphoreType.DMA] * (2 * NBUF)
    ),
)
def _sc_scatter(src_hbm, dst_hbm, alpha_hbm, x_hbm, p0_hbm, p1_hbm,
                idx_d, acc, *rest):
  rows = rest[:NBUF]
  abuf = rest[NBUF:2 * NBUF]
  isbuf = rest[2 * NBUF:3 * NBUF]
  gsem = rest[3 * NBUF:4 * NBUF]
  csem = rest[4 * NBUF:]
  cid = lax.axis_index("c")
  sid = lax.axis_index("s")
  wid = sid * NC + cid
  base = wid * EPT

  pltpu.sync_copy(dst_hbm.at[pl.ds(base, EPT)], idx_d)

  def zrow(k, c2):
    for v in range(VPR):
      rows[0][k, pl.ds(v * LANES, LANES)] = jnp.zeros((LANES,), f32)
    return c2

  lax.fori_loop(0, CS, zrow, 0)

  def zchunk(ci, c2):
    @pl.when(ci % NS == sid)
    def _():
      pltpu.sync_copy(rows[0], acc.at[pl.ds(ci * CS, CS)])
    return c2

  lax.fori_loop(0, NZ, zchunk, 0)
  plsc.subcore_barrier()

  def g_issue(c, s):
    pltpu.async_copy(x_hbm.at[idx_d.at[pl.ds(c * CS, CS)]], rows[s], gsem[s])
    pltpu.async_copy(alpha_hbm.at[pl.ds(base + c * CS, CS)],
                     abuf[s].at[pl.ds(0, CS)], gsem[s])
    pltpu.async_copy(src_hbm.at[pl.ds(base + c * CS, CS)], isbuf[s], gsem[s])

  def g_wait(s):
    pltpu.make_async_copy(x_hbm.at[pl.ds(0, CS)], rows[s], gsem[s]).wait()
    pltpu.make_async_copy(alpha_hbm.at[pl.ds(0, CS)],
                          abuf[s].at[pl.ds(0, CS)], gsem[s]).wait()
    pltpu.make_async_copy(src_hbm.at[pl.ds(0, CS)], isbuf[s], gsem[s]).wait()

  def c_issue(s):
    pltpu.async_copy(rows[s], acc.at[isbuf[s]], csem[s], add=True)

  def c_wait(s):
    pltpu.make_async_copy(rows[s], acc.at[pl.ds(0, CS)], csem[s]).wait()

  for c in range(NBUF - 1):
    g_issue(c, c)

  def outer(jj, carry):
    for b in range(NBUF):
      j = jj * NBUF + b

      @pl.when(j + 4 < NCHS)
      def _():
        s2 = (b + 4) % NBUF

        @pl.when(j >= 1)
        def _():
          c_wait(s2)

        g_issue(j + 4, s2)

      g_wait(b)

      def scale16(q, c2):
        nav = -abuf[b][pl.ds(q * LANES, LANES)]
        for t in range(LANES):
          a = nav[t]
          k = q * LANES + t
          for v in range(VPR):
            sl = pl.ds(v * LANES, LANES)
            rows[b][k, sl] = rows[b][k, sl] * a
        return c2

      lax.fori_loop(0, 2, scale16, 0)
      nav2 = -abuf[b][pl.ds(32, LANES)]
      for t in range(8):
        a = nav2[t]
        for v in range(VPR):
          sl = pl.ds(v * LANES, LANES)
          rows[b][32 + t, sl] = rows[b][32 + t, sl] * a
      c_issue(b)
    return carry

  lax.fori_loop(0, NCHS // NBUF, outer, 0)
  for s in range(NBUF):
    c_wait(s)
  plsc.subcore_barrier()

  def drain(ci, c2):
    @pl.when(ci % NS == sid)
    def _():
      sl = pl.ds(ci * CS, CS)

      @pl.when(cid == 0)
      def _():
        pltpu.sync_copy(acc.at[sl], p0_hbm.at[sl])

      @pl.when(cid == 1)
      def _():
        pltpu.sync_copy(acc.at[sl], p1_hbm.at[sl])
    return c2

  lax.fori_loop(0, NZ, drain, 0)


RB = 2000
NRB = N // RB
EB = 8000
NEB = E // EB
SRows = 2500


def _p1_body(x_ref, w1t_ref, w2t_ref, xs_ref, xdn_ref):
  xb = x_ref[...]
  xs_ref[...] = jnp.dot(xb, w1t_ref[...], preferred_element_type=f32)
  xdn_ref[...] = jnp.dot(xb, w2t_ref[...], preferred_element_type=f32)


_p1 = pl.pallas_call(
    _p1_body,
    grid=(NRB,),
    in_specs=[
        pl.BlockSpec((RB, D), lambda i: (i, 0)),
        pl.BlockSpec((D, H), lambda i: (0, 0)),
        pl.BlockSpec((D, H), lambda i: (0, 0)),
    ],
    out_specs=[
        pl.BlockSpec((RB, H), lambda i: (i, 0)),
        pl.BlockSpec((RB, H), lambda i: (i, 0)),
    ],
    out_shape=[
        jax.ShapeDtypeStruct((N, H), f32),
        jax.ShapeDtypeStruct((N, H), f32),
    ],
)


def _p2_body(g_ref, ea_ref, wht_ref, bias_ref, we_ref, s_ref):
  z = (g_ref[...]
       + jnp.dot(ea_ref[...], wht_ref[...], preferred_element_type=f32)
       + bias_ref[...])
  z = jnp.maximum(z, 0.0)
  s = jnp.sum(z * we_ref[...], axis=1)
  s_ref[...] = s.reshape(1, 1, EB)


_p2 = pl.pallas_call(
    _p2_body,
    grid=(NEB,),
    in_specs=[
        pl.BlockSpec((EB, H), lambda i: (i, 0)),
        pl.BlockSpec((EB, DE), lambda i: (i, 0)),
        pl.BlockSpec((DE, H), lambda i: (0, 0)),
        pl.BlockSpec((1, H), lambda i: (0, 0)),
        pl.BlockSpec((1, H), lambda i: (0, 0)),
    ],
    out_specs=pl.BlockSpec((1, 1, EB), lambda i: (i, 0, 0)),
    out_shape=jax.ShapeDtypeStruct((NEB, 1, EB), f32),
)


def _p3_body(s_ref, a_ref):
  s = s_ref[...]
  m = jnp.max(s)
  e = jnp.exp(s - m)
  a_ref[...] = e / jnp.sum(e)


_p3 = pl.pallas_call(
    _p3_body,
    out_shape=jax.ShapeDtypeStruct((SRows, 128), f32),
)


def _p4_body(x_ref, p0_ref, p1_ref, ftt_ref, ftb_ref, o_ref):
  o1 = x_ref[...] + p0_ref[...] + p1_ref[...]
  o_ref[...] = (o1 + jnp.dot(o1, ftt_ref[...], preferred_element_type=f32)
                + ftb_ref[...])


_p4 = pl.pallas_call(
    _p4_body,
    grid=(NRB,),
    in_specs=[
        pl.BlockSpec((RB, D), lambda i: (i, 0)),
        pl.BlockSpec((RB, D), lambda i: (i, 0)),
        pl.BlockSpec((RB, D), lambda i: (i, 0)),
        pl.BlockSpec((D, D), lambda i: (0, 0)),
        pl.BlockSpec((1, D), lambda i: (0, 0)),
    ],
    out_specs=pl.BlockSpec((RB, D), lambda i: (i, 0)),
    out_shape=jax.ShapeDtypeStruct((N, D), f32),
)


def kernel(x, edge_index, edge_attr, Wh_w, Wh_b, Wn_w, Wn_b, w_e, ft_w, ft_b):
  src = edge_index[0]
  dst = edge_index[1]
  w1t = Wn_w[:, :D].T
  w2t = Wn_w[:, D:].T
  wht = Wh_w.T
  bias = (Wh_b + Wn_b).reshape(1, H)
  ftt = ft_w.T
  ftb = ft_b.reshape(1, D)

  xs, xdn = _p1(x, w1t, w2t)
  g = _sc_gather_combine(src, dst, xs, xdn)
  s = _p2(g, edge_attr, wht, bias, w_e.reshape(1, H))
  alpha2 = _p3(s.reshape(SRows, 128))
  alpha = alpha2.reshape(E)
  part0, part1 = _sc_scatter(src, dst, alpha, x)
  out = _p4(x, part0, part1, ftt, ftb)
  return out, alpha
